# baseline (speedup 1.0000x reference)
import jax
import jax.numpy as jnp
from jax import lax
from jax.experimental import pallas as pl
from jax.experimental.pallas import tpu as pltpu
from jax.experimental.pallas import tpu_sc as plsc

B = 16384
D = 32
HALF = D // 2
GRP = 4
V = 1000000
NC = 2
NS = 16
NW = NC * NS
B_PER_W = B // NW
IDX_CHUNK = 128
N_CHUNKS = B_PER_W // IDX_CHUNK
BLK_I = 65536
SUB = BLK_I // GRP
N_BLK = (V + BLK_I - 1) // BLK_I
VP = N_BLK * SUB


def _pack16(x):
    xi = lax.bitcast_convert_type(x, jnp.uint32)
    lsb = (xi >> 16) & jnp.uint32(1)
    xr = (xi + jnp.uint32(0x7FFF) + lsb) >> 16
    packed = (xr[:HALF] << 16) | xr[HALF:]
    return lax.bitcast_convert_type(packed, jnp.float32)


def _repack_body(xg_ref, xm_ref, out_ref):
    pg = _pack16(xg_ref[...])
    pm = _pack16(xm_ref[...])
    for k in range(SUB // 128):
        xv = jnp.concatenate(
            [p[:, c * SUB + k * 128:c * SUB + k * 128 + 128]
             for c in range(GRP) for p in (pg, pm)], axis=0)
        out_ref[k * 128:(k + 1) * 128, :] = jnp.transpose(xv)


def _repack(tg_t, tm_t):
    return pl.pallas_call(
        _repack_body,
        grid=(N_BLK,),
        in_specs=[pl.BlockSpec((D, BLK_I), lambda i: (0, i)),
                  pl.BlockSpec((D, BLK_I), lambda i: (0, i))],
        out_specs=pl.BlockSpec((SUB, 128), lambda i: (i, 0)),
        out_shape=jax.ShapeDtypeStruct((VP, 128), jnp.float32),
    )(tg_t, tm_t)


def _sc_gather_body(qids_hbm, tbl_hbm, out_hbm, idx_v, rows_v, sem):
    n_chunks, b_per_w = idx_v.shape[0], rows_v.shape[0]
    wid = lax.axis_index("s") * NC + lax.axis_index("c")
    pltpu.sync_copy(qids_hbm.at[wid], idx_v)
    copies = []
    for j in range(n_chunks):
        rows = pl.ds(j * IDX_CHUNK, IDX_CHUNK)
        copies.append(pltpu.async_copy(tbl_hbm.at[idx_v.at[j]],
                                       rows_v.at[rows], sem))
    for c in copies:
        c.wait()
    pltpu.sync_copy(rows_v, out_hbm.at[pl.ds(wid * b_per_w, b_per_w)])


def _sc_gather(qids, tbl):
    n_chunks = qids.shape[1]
    b_per_w = n_chunks * IDX_CHUNK
    mesh = plsc.VectorSubcoreMesh(core_axis_name="c", subcore_axis_name="s")
    run = pl.kernel(
        _sc_gather_body,
        out_type=jax.ShapeDtypeStruct((NW * b_per_w, 128), jnp.float32),
        mesh=mesh,
        scratch_types=[
            pltpu.VMEM((n_chunks, IDX_CHUNK), jnp.int32),
            pltpu.VMEM((b_per_w, 128), jnp.float32),
            pltpu.SemaphoreType.DMA,
        ],
    )
    return run(qids, tbl)


def _unpack_sel(x128, sel):
    xi = lax.bitcast_convert_type(x128, jnp.uint32)
    lane_w = lax.broadcasted_iota(jnp.int32, xi.shape, 1) // D
    z = jnp.where(lane_w == sel, xi, jnp.uint32(0))
    z = z[:, :64] | z[:, 64:]
    w = z[:, :D] | z[:, D:]
    hi = lax.bitcast_convert_type(w & jnp.uint32(0xFFFF0000), jnp.float32)
    lo = lax.bitcast_convert_type(w << 16, jnp.float32)
    g = jnp.concatenate([hi[:, :HALF], lo[:, :HALF]], axis=1)
    m = jnp.concatenate([hi[:, HALF:], lo[:, HALF:]], axis=1)
    return g, m


def _mlp_body(usel_ref, isel_ref, gu_ref, gi_ref,
              W1u_ref, W1i_ref, b1_ref, W2_ref, b2_ref, W3_ref, b3_ref,
              Wfg_ref, Wfh_ref, bf_ref, out_ref):
    f32 = jnp.float32
    ug, um = _unpack_sel(gu_ref[...], usel_ref[...])
    ig, im = _unpack_sel(gi_ref[...], isel_ref[...])
    gmf = ug * ig
    h = jnp.dot(um, W1u_ref[...], preferred_element_type=f32)
    h += jnp.dot(im, W1i_ref[...], preferred_element_type=f32)
    h = jnp.maximum(h + b1_ref[...], 0.0)
    h = jnp.maximum(jnp.dot(h, W2_ref[...], preferred_element_type=f32)
                    + b2_ref[...], 0.0)
    h = jnp.maximum(jnp.dot(h, W3_ref[...], preferred_element_type=f32)
                    + b3_ref[...], 0.0)
    out = jnp.dot(gmf, Wfg_ref[...], preferred_element_type=f32)
    out += jnp.dot(h, Wfh_ref[...], preferred_element_type=f32)
    out_ref[...] = out + bf_ref[...]


def _mlp(usel, isel, gu, gi, W1, b1, W2, b2, W3, b3, Wf, bf):
    bh = gu.shape[0]
    n_blk = 2
    blk = bh // n_blk
    full = lambda shape: pl.BlockSpec(shape, lambda i: (0, 0))
    row = lambda w: pl.BlockSpec((blk, w), lambda i: (i, 0))
    grid_spec = pl.GridSpec(
        grid=(n_blk,),
        in_specs=[
            row(1), row(1),
            row(128), row(128),
            full((D, 64)), full((D, 64)), full((1, 64)),
            full((64, 32)), full((1, 32)),
            full((32, 16)), full((1, 16)),
            full((D, 1)), full((16, 1)), full((1, 1)),
        ],
        out_specs=pl.BlockSpec((blk, 1), lambda i: (i, 0)),
    )
    return pl.pallas_call(
        _mlp_body,
        grid_spec=grid_spec,
        out_shape=jax.ShapeDtypeStruct((bh, 1), jnp.float32),
    )(usel, isel, gu, gi,
      W1[:D], W1[D:], b1.reshape(1, -1),
      W2, b2.reshape(1, -1),
      W3, b3.reshape(1, -1),
      Wf[:D], Wf[D:], bf.reshape(1, 1))


def kernel(user_ids, item_ids, user_gmf, item_gmf, user_mlp, item_mlp,
           W1, b1, W2, b2, W3, b3, Wf, bf):
    uids = user_ids.astype(jnp.int32)
    iids = item_ids.astype(jnp.int32)
    uq = (uids // BLK_I) * SUB + uids % SUB
    iq = (iids // BLK_I) * SUB + iids % SUB
    usel = (uids // SUB) % GRP
    isel = (iids // SUB) % GRP
    pu = _repack(user_gmf.T, user_mlp.T)
    pi = _repack(item_gmf.T, item_mlp.T)
    bh = B // 2
    outs = []
    for h in range(2):
        s = slice(h * bh, (h + 1) * bh)
        gu = _sc_gather(uq[s].reshape(NW, -1, IDX_CHUNK), pu)
        gi = _sc_gather(iq[s].reshape(NW, -1, IDX_CHUNK), pi)
        outs.append(_mlp(usel[s].reshape(bh, 1), isel[s].reshape(bh, 1),
                         gu, gi, W1, b1, W2, b2, W3, b3, Wf, bf))
    return jnp.concatenate(outs, axis=0)


# speedup vs baseline: 1.0391x; 1.0391x over previous
import jax
import jax.numpy as jnp
from jax import lax
from jax.experimental import pallas as pl
from jax.experimental.pallas import tpu as pltpu
from jax.experimental.pallas import tpu_sc as plsc

B = 16384
D = 32
HALF = D // 2
GRP = 4
V = 1000000
NC = 2
NS = 16
NW = NC * NS
B_PER_W = B // NW
IDX_CHUNK = 128
N_CHUNKS = B_PER_W // IDX_CHUNK
BLK_I = 65536
SUB = BLK_I // GRP
N_BLK = (V + BLK_I - 1) // BLK_I
VP = N_BLK * SUB


def _pack16(x):
    xi = lax.bitcast_convert_type(x, jnp.uint32)
    lsb = (xi >> 16) & jnp.uint32(1)
    xr = (xi + jnp.uint32(0x7FFF) + lsb) >> 16
    packed = (xr[:HALF] << 16) | xr[HALF:]
    return lax.bitcast_convert_type(packed, jnp.float32)


def _repack_body(xg_ref, xm_ref, out_ref):
    pg = _pack16(xg_ref[...])
    pm = _pack16(xm_ref[...])
    for k in range(SUB // 128):
        xv = jnp.concatenate(
            [p[:, c * SUB + k * 128:c * SUB + k * 128 + 128]
             for c in range(GRP) for p in (pg, pm)], axis=0)
        out_ref[k * 128:(k + 1) * 128, :] = jnp.transpose(xv)


def _repack(tg_t, tm_t):
    return pl.pallas_call(
        _repack_body,
        grid=(N_BLK,),
        in_specs=[pl.BlockSpec((D, BLK_I), lambda i: (0, i)),
                  pl.BlockSpec((D, BLK_I), lambda i: (0, i))],
        out_specs=pl.BlockSpec((SUB, 128), lambda i: (i, 0)),
        out_shape=jax.ShapeDtypeStruct((VP, 128), jnp.float32),
    )(tg_t, tm_t)


def _sc_gather_body(qids_hbm, tbl_hbm, out_hbm, idx_v, rows_v, sem):
    wid = lax.axis_index("s") * NC + lax.axis_index("c")
    pltpu.sync_copy(qids_hbm.at[wid], idx_v)
    copies = []
    for j in range(N_CHUNKS):
        rows = pl.ds(j * IDX_CHUNK, IDX_CHUNK)
        copies.append(pltpu.async_copy(tbl_hbm.at[idx_v.at[j]],
                                       rows_v.at[rows], sem))
    for c in copies:
        c.wait()
    pltpu.sync_copy(rows_v, out_hbm.at[pl.ds(wid * B_PER_W, B_PER_W)])


def _sc_gather(qids, tbl):
    mesh = plsc.VectorSubcoreMesh(core_axis_name="c", subcore_axis_name="s")
    run = pl.kernel(
        _sc_gather_body,
        out_type=jax.ShapeDtypeStruct((B, 128), jnp.float32),
        mesh=mesh,
        scratch_types=[
            pltpu.VMEM((N_CHUNKS, IDX_CHUNK), jnp.int32),
            pltpu.VMEM((B_PER_W, 128), jnp.float32),
            pltpu.SemaphoreType.DMA,
        ],
    )
    return run(qids, tbl)


def _unpack_sel(x128, sel):
    xi = lax.bitcast_convert_type(x128, jnp.uint32)
    lane_w = lax.broadcasted_iota(jnp.int32, xi.shape, 1) // D
    z = jnp.where(lane_w == sel, xi, jnp.uint32(0))
    z = z[:, :64] | z[:, 64:]
    w = z[:, :D] | z[:, D:]
    hi = lax.bitcast_convert_type(w & jnp.uint32(0xFFFF0000), jnp.float32)
    lo = lax.bitcast_convert_type(w << 16, jnp.float32)
    g = jnp.concatenate([hi[:, :HALF], lo[:, :HALF]], axis=1)
    m = jnp.concatenate([hi[:, HALF:], lo[:, HALF:]], axis=1)
    return g, m


def _mlp_body(usel_ref, isel_ref, gu_ref, gi_ref,
              W1u_ref, W1i_ref, b1_ref, W2_ref, b2_ref, W3_ref, b3_ref,
              Wfg_ref, Wfh_ref, bf_ref, out_ref):
    f32 = jnp.float32
    ug, um = _unpack_sel(gu_ref[...], usel_ref[...])
    ig, im = _unpack_sel(gi_ref[...], isel_ref[...])
    gmf = ug * ig
    h = jnp.dot(um, W1u_ref[...], preferred_element_type=f32)
    h += jnp.dot(im, W1i_ref[...], preferred_element_type=f32)
    h = jnp.maximum(h + b1_ref[...], 0.0)
    h = jnp.maximum(jnp.dot(h, W2_ref[...], preferred_element_type=f32)
                    + b2_ref[...], 0.0)
    h = jnp.maximum(jnp.dot(h, W3_ref[...], preferred_element_type=f32)
                    + b3_ref[...], 0.0)
    out = jnp.dot(gmf, Wfg_ref[...], preferred_element_type=f32)
    out += jnp.dot(h, Wfh_ref[...], preferred_element_type=f32)
    out_ref[...] = out + bf_ref[...]


def _mlp(usel, isel, gu, gi, W1, b1, W2, b2, W3, b3, Wf, bf):
    n_blk = 4
    blk = B // n_blk
    full = lambda shape: pl.BlockSpec(shape, lambda i: (0, 0))
    row = lambda w: pl.BlockSpec((blk, w), lambda i: (i, 0))
    grid_spec = pl.GridSpec(
        grid=(n_blk,),
        in_specs=[
            row(1), row(1),
            row(128), row(128),
            full((D, 64)), full((D, 64)), full((1, 64)),
            full((64, 32)), full((1, 32)),
            full((32, 16)), full((1, 16)),
            full((D, 1)), full((16, 1)), full((1, 1)),
        ],
        out_specs=pl.BlockSpec((blk, 1), lambda i: (i, 0)),
    )
    return pl.pallas_call(
        _mlp_body,
        grid_spec=grid_spec,
        out_shape=jax.ShapeDtypeStruct((B, 1), jnp.float32),
    )(usel, isel, gu, gi,
      W1[:D], W1[D:], b1.reshape(1, -1),
      W2, b2.reshape(1, -1),
      W3, b3.reshape(1, -1),
      Wf[:D], Wf[D:], bf.reshape(1, 1))


def kernel(user_ids, item_ids, user_gmf, item_gmf, user_mlp, item_mlp,
           W1, b1, W2, b2, W3, b3, Wf, bf):
    uids = user_ids.astype(jnp.int32)
    iids = item_ids.astype(jnp.int32)
    uq = ((uids // BLK_I) * SUB + uids % SUB).reshape(NW, N_CHUNKS, IDX_CHUNK)
    iq = ((iids // BLK_I) * SUB + iids % SUB).reshape(NW, N_CHUNKS, IDX_CHUNK)
    usel = ((uids // SUB) % GRP).reshape(B, 1)
    isel = ((iids // SUB) % GRP).reshape(B, 1)
    gu = _sc_gather(uq, _repack(user_gmf.T, user_mlp.T))
    gi = _sc_gather(iq, _repack(item_gmf.T, item_mlp.T))
    return _mlp(usel, isel, gu, gi, W1, b1, W2, b2, W3, b3, Wf, bf)
